# Initial kernel scaffold; baseline (speedup 1.0000x reference)
import functools

import jax
import jax.numpy as jnp
from jax import lax
from jax.experimental import pallas as pl
from jax.experimental.pallas import tpu as pltpu
from jax.experimental.pallas import tpu_sc as plsc

NEG_SLOPE = 0.2
LANES = 16
CHUNK = 128
NC, NS = 2, 16
NW = NC * NS
ACC_W = 144


def _prep_body(x_ref, w_ref, asrc_ref, adst_ref, h_ref, sd_ref):
    n = x_ref.shape[0]
    n_ext = sd_ref.shape[1]
    h = jnp.dot(x_ref[...], w_ref[...], preferred_element_type=jnp.float32)
    h_ref[...] = h
    s = jnp.dot(h, asrc_ref[...].reshape(-1, 1),
                preferred_element_type=jnp.float32)[:, 0]
    d = jnp.dot(h, adst_ref[...].reshape(-1, 1),
                preferred_element_type=jnp.float32)[:, 0]
    c = jnp.max(s) + jnp.max(d)
    c = jnp.where(c > 0, c, NEG_SLOPE * c)
    pad = jnp.zeros((n_ext - n,), jnp.float32)
    s_ext = jnp.concatenate([s, pad])
    d_ext = jnp.concatenate([d, pad])
    idx = lax.broadcasted_iota(jnp.int32, (n_ext,), 0)
    s_ext = jnp.where(idx == n, c, s_ext)
    sd_ref[...] = jnp.stack([s_ext, d_ext])


def _final_body(u_ref, bias_ref, out_ref):
    u = u_ref[0] + u_ref[1]
    num = u[:, :128]
    den = u[:, 128:129]
    out_ref[...] = jnp.tanh(num / den + bias_ref[...][None, :])


def _sc_body(n_nodes, n_total, per_w, n_chunks,
             h_hbm, sd_hbm, src_hbm, dst_hbm, zero_hbm, out_hbm,
             u_sh, s_v, d_v, srcv, dstv, hrows, prod, eev, sem_g, sem_s):
    cid = lax.axis_index("c")
    sid = lax.axis_index("s")
    wid = sid * NC + cid
    rows_per_tile = n_nodes // NS

    pltpu.sync_copy(zero_hbm.at[pl.ds(sid * rows_per_tile, rows_per_tile)],
                    u_sh.at[pl.ds(sid * rows_per_tile, rows_per_tile)])
    pltpu.sync_copy(sd_hbm.at[0], s_v)
    pltpu.sync_copy(sd_hbm.at[1], d_v)
    plsc.subcore_barrier()

    cvec = plsc.load_gather(s_v, [jnp.full((LANES,), n_nodes, jnp.int32)])

    def chunk_body(c, carry):
        base = pl.multiple_of(wid * per_w + c * CHUNK, CHUNK)
        pltpu.sync_copy(src_hbm.at[pl.ds(base, CHUNK)], srcv)
        pltpu.sync_copy(dst_hbm.at[pl.ds(base, CHUNK)], dstv)
        pltpu.async_copy(h_hbm.at[srcv], hrows, sem_g).wait()

        def group_body(g, carry2):
            sidx = srcv[pl.ds(g * LANES, LANES)]
            didx = dstv[pl.ds(g * LANES, LANES)]
            e = plsc.load_gather(s_v, [sidx]) + plsc.load_gather(d_v, [didx])
            e = jnp.where(e > 0, e, NEG_SLOPE * e) - cvec
            gi = base + g * LANES + lax.iota(jnp.int32, LANES)
            eev[pl.ds(g * LANES, LANES)] = jnp.where(
                gi < n_total, jnp.exp(e), 0.0)
            return carry2

        lax.fori_loop(0, CHUNK // LANES, group_body, 0)

        def edge_body(k, carry2):
            spl = plsc.load_gather(eev, [jnp.zeros((LANES,), jnp.int32) + k])
            for cc in range(8):
                prod[k, pl.ds(cc * LANES, LANES)] = (
                    hrows[k, pl.ds(cc * LANES, LANES)] * spl)
            prod[k, pl.ds(128, LANES)] = spl
            return carry2

        lax.fori_loop(0, CHUNK, edge_body, 0)
        pltpu.async_copy(prod, u_sh.at[dstv], sem_s, add=True).wait()
        return carry

    lax.fori_loop(0, n_chunks, chunk_body, 0)
    plsc.subcore_barrier()

    @pl.when(sid == 0)
    def _():
        pltpu.sync_copy(u_sh, out_hbm.at[cid])


def kernel(x, edge_index, W, att_src, att_dst, bias):
    n, _ = x.shape
    dout = W.shape[1]
    e = edge_index.shape[1]
    n_total = e + n
    epad = ((n_total + NW * CHUNK - 1) // (NW * CHUNK)) * (NW * CHUNK)
    per_w = epad // NW
    n_chunks = per_w // CHUNK
    n_ext = n + LANES

    loops = jnp.arange(n, dtype=edge_index.dtype)
    padz = jnp.zeros((epad - n_total,), edge_index.dtype)
    src = jnp.concatenate([edge_index[0], loops, padz])
    dst = jnp.concatenate([edge_index[1], loops, padz])

    h, sd = pl.pallas_call(
        _prep_body,
        out_shape=(
            jax.ShapeDtypeStruct((n, dout), jnp.float32),
            jax.ShapeDtypeStruct((2, n_ext), jnp.float32),
        ),
    )(x, W, att_src, att_dst)

    zero = jnp.zeros((n, ACC_W), jnp.float32)

    mesh = plsc.VectorSubcoreMesh(
        core_axis_name="c", subcore_axis_name="s",
        num_cores=NC, num_subcores=NS)
    sc_fn = pl.kernel(
        functools.partial(_sc_body, n, n_total, per_w, n_chunks),
        out_type=jax.ShapeDtypeStruct((NC, n, ACC_W), jnp.float32),
        mesh=mesh,
        scratch_types=[
            pltpu.VMEM_SHARED((n, ACC_W), jnp.float32),
            pltpu.VMEM((n_ext,), jnp.float32),
            pltpu.VMEM((n_ext,), jnp.float32),
            pltpu.VMEM((CHUNK,), jnp.int32),
            pltpu.VMEM((CHUNK,), jnp.int32),
            pltpu.VMEM((CHUNK, 128), jnp.float32),
            pltpu.VMEM((CHUNK, ACC_W), jnp.float32),
            pltpu.VMEM((CHUNK,), jnp.float32),
            pltpu.SemaphoreType.DMA,
            pltpu.SemaphoreType.DMA,
        ],
    )
    upart = sc_fn(h, sd, src, dst, zero)

    out = pl.pallas_call(
        _final_body,
        out_shape=jax.ShapeDtypeStruct((n, dout), jnp.float32),
    )(upart, bias)
    return out


# speedup vs baseline: 19.3510x; 19.3510x over previous
import dataclasses
import functools

import jax
import jax.numpy as jnp
from jax import lax
from jax.experimental import pallas as pl
from jax.experimental.pallas import tpu as pltpu
from jax.experimental.pallas import tpu_sc as plsc

NEG_SLOPE = 0.2
LANES = 16
CHUNK = 128
NC, NS = 2, 16
NW = NC * NS


def _prep_body(x_ref, w_ref, asrc_ref, adst_ref, h_ref, sd_ref):
    n = x_ref.shape[0]
    n_ext = sd_ref.shape[1]
    h = jnp.dot(x_ref[...], w_ref[...], preferred_element_type=jnp.float32)
    h_ref[...] = h
    s = jnp.dot(h, asrc_ref[...].reshape(-1, 1),
                preferred_element_type=jnp.float32)[:, 0]
    d = jnp.dot(h, adst_ref[...].reshape(-1, 1),
                preferred_element_type=jnp.float32)[:, 0]
    c = jnp.max(s) + jnp.max(d)
    c = jnp.where(c > 0, c, NEG_SLOPE * c)
    pad = jnp.zeros((n_ext - n,), jnp.float32)
    s_ext = jnp.concatenate([s, pad])
    d_ext = jnp.concatenate([d, pad])
    idx = lax.broadcasted_iota(jnp.int32, (n_ext,), 0)
    s_ext = jnp.where(idx == n, c, s_ext)
    sd_ref[...] = jnp.stack([s_ext, d_ext])


def _final_body(u_ref, d_ref, bias_ref, out_ref):
    num = u_ref[0] + u_ref[1]
    den = jnp.sum(d_ref[...], axis=0)
    out_ref[...] = jnp.tanh(num / den[:, None] + bias_ref[...][None, :])


def _sc_body(n_nodes, n_total, per_w, n_chunks,
             h_hbm, sd_hbm, src_hbm, dst_hbm, zero_hbm, out_hbm, den_hbm,
             u_sh, s_v, d_v, denl, srcv, dstv, hrows, eev,
             sem_g, sem_s):
    cid = lax.axis_index("c")
    sid = lax.axis_index("s")
    wid = sid * NC + cid
    rows_per_tile = (n_nodes // NS) // 8 * 8
    rem_base = rows_per_tile * NS
    rem = n_nodes - rem_base

    off = pl.multiple_of(sid * rows_per_tile, 8)
    pltpu.sync_copy(zero_hbm.at[pl.ds(off, rows_per_tile)],
                    u_sh.at[pl.ds(off, rows_per_tile)])
    if rem:
        @pl.when(sid == 0)
        def _():
            pltpu.sync_copy(zero_hbm.at[pl.ds(rem_base, rem)],
                            u_sh.at[pl.ds(rem_base, rem)])
    pltpu.sync_copy(sd_hbm.at[0], s_v)
    pltpu.sync_copy(sd_hbm.at[1], d_v)

    def zden(j, carry):
        denl[pl.ds(j * LANES, LANES)] = jnp.zeros((LANES,), jnp.float32)
        return carry
    lax.fori_loop(0, (denl.shape[0] + LANES - 1) // LANES, zden, 0)
    plsc.subcore_barrier()

    cvec = plsc.load_gather(s_v, [jnp.full((LANES,), n_nodes, jnp.int32)])
    lane0 = lax.iota(jnp.int32, LANES) == 0

    def chunk_body(c, carry):
        base = pl.multiple_of(wid * per_w + c * CHUNK, CHUNK)
        pltpu.sync_copy(src_hbm.at[pl.ds(base, CHUNK)], srcv)
        pltpu.sync_copy(dst_hbm.at[pl.ds(base, CHUNK)], dstv)
        pltpu.async_copy(h_hbm.at[srcv], hrows, sem_g).wait()

        def group_body(g, carry2):
            sidx = srcv[pl.ds(g * LANES, LANES)]
            didx = dstv[pl.ds(g * LANES, LANES)]
            e = plsc.load_gather(s_v, [sidx]) + plsc.load_gather(d_v, [didx])
            e = jnp.where(e > 0, e, NEG_SLOPE * e) - cvec
            gi = base + g * LANES + lax.iota(jnp.int32, LANES)
            eev[pl.ds(g * LANES, LANES)] = jnp.where(
                gi < n_total, jnp.exp(e), 0.0)
            return carry2

        lax.fori_loop(0, CHUNK // LANES, group_body, 0)

        def edge_body(k, carry2):
            kidx = jnp.zeros((LANES,), jnp.int32) + k
            spl = plsc.load_gather(eev, [kidx])
            for cc in range(8):
                hrows[k, pl.ds(cc * LANES, LANES)] = (
                    hrows[k, pl.ds(cc * LANES, LANES)] * spl)
            dsp = plsc.load_gather(dstv, [kidx])
            dcur = plsc.load_gather(denl, [dsp])
            plsc.store_scatter(denl, [dsp], dcur + spl, mask=lane0)
            return carry2

        lax.fori_loop(0, CHUNK, edge_body, 0)
        pltpu.async_copy(hrows, u_sh.at[dstv], sem_s, add=True).wait()
        return carry

    lax.fori_loop(0, n_chunks, chunk_body, 0)
    pltpu.sync_copy(denl, den_hbm.at[wid])
    plsc.subcore_barrier()

    @pl.when(sid == 0)
    def _():
        pltpu.sync_copy(u_sh, out_hbm.at[cid])


def kernel(x, edge_index, W, att_src, att_dst, bias):
    n, _ = x.shape
    dout = W.shape[1]
    e = edge_index.shape[1]
    n_total = e + n
    epad = ((n_total + NW * CHUNK - 1) // (NW * CHUNK)) * (NW * CHUNK)
    per_w = epad // NW
    n_chunks = per_w // CHUNK
    n_ext = n + LANES

    loops = jnp.arange(n, dtype=edge_index.dtype)
    padz = jnp.zeros((epad - n_total,), edge_index.dtype)
    src = jnp.concatenate([edge_index[0], loops, padz])
    dst = jnp.concatenate([edge_index[1], loops, padz])

    h, sd = pl.pallas_call(
        _prep_body,
        out_shape=(
            jax.ShapeDtypeStruct((n, dout), jnp.float32),
            jax.ShapeDtypeStruct((2, n_ext), jnp.float32),
        ),
    )(x, W, att_src, att_dst)

    zero = jnp.zeros((n, dout), jnp.float32)

    mesh = plsc.VectorSubcoreMesh(
        core_axis_name="c", subcore_axis_name="s",
        num_cores=NC, num_subcores=NS)
    cp = pltpu.CompilerParams()
    if "needs_layout_passes" in pltpu.CompilerParams.__dataclass_fields__:
        cp = dataclasses.replace(cp, needs_layout_passes=False)
    sc_fn = pl.kernel(
        functools.partial(_sc_body, n, n_total, per_w, n_chunks),
        out_type=(jax.ShapeDtypeStruct((NC, n, dout), jnp.float32),
                  jax.ShapeDtypeStruct((NW, n), jnp.float32)),
        mesh=mesh,
        compiler_params=cp,
        scratch_types=[
            pltpu.VMEM_SHARED((n, dout), jnp.float32),
            pltpu.VMEM((n_ext,), jnp.float32),
            pltpu.VMEM((n_ext,), jnp.float32),
            pltpu.VMEM((n,), jnp.float32),
            pltpu.VMEM((CHUNK,), jnp.int32),
            pltpu.VMEM((CHUNK,), jnp.int32),
            pltpu.VMEM((CHUNK, 128), jnp.float32),
            pltpu.VMEM((CHUNK,), jnp.float32),
            pltpu.SemaphoreType.DMA,
            pltpu.SemaphoreType.DMA,
        ],
    )
    upart, dpart = sc_fn(h, sd, src, dst, zero)

    out = pl.pallas_call(
        _final_body,
        out_shape=jax.ShapeDtypeStruct((n, dout), jnp.float32),
    )(upart, dpart, bias)
    return out
